# baseline (speedup 1.0000x reference)
import functools

import jax
import jax.numpy as jnp
from jax import lax
from jax.experimental import pallas as pl
from jax.experimental.pallas import tpu as pltpu
from jax.experimental.pallas import tpu_sc as plsc

N = 10000
E = 320000
D = 128
NB = 32
OUT = 64
CUTOFF = 5.0

NC = 2
NS = 16
NW = NC * NS
CH = 80
NACC = 10240
NPAD = NW * CH * 4

_MESH = dict(core_axis_name="c", subcore_axis_name="s")
_NOLAYOUT = pltpu.CompilerParams(needs_layout_passes=False)


def _mul_rows(dst_ref, a_ref, b_ref, n_rows):
    @plsc.parallel_loop(0, n_rows, 1, unroll=4)
    def row(r):
        for c8 in range(D // 16):
            s = pl.ds(c8 * 16, 16)
            dst_ref[r, s] = a_ref[r, s] * b_ref[r, s]


def _add_rows(dst_ref, a_ref, b_ref, n_rows):
    @plsc.parallel_loop(0, n_rows, 1, unroll=4)
    def row(r):
        for c8 in range(D // 16):
            s = pl.ds(c8 * 16, 16)
            dst_ref[r, s] = a_ref[r, s] + b_ref[r, s]


def _make_sc_geo_embed():
    per_w = E // NW
    GCH = 2000
    g_chunks = per_w // GCH
    iters = GCH // 16
    e_chunks = NPAD // (NW * CH)

    sd = jax.ShapeDtypeStruct((E,), jnp.float32)

    @functools.partial(
        pl.kernel,
        out_type=(sd, sd, sd, jax.ShapeDtypeStruct((NPAD, D), jnp.float32)),
        mesh=plsc.VectorSubcoreMesh(**_MESH),
        compiler_params=_NOLAYOUT,
        scratch_types=[
            pltpu.VMEM((N,), jnp.float32),
            pltpu.VMEM((N,), jnp.float32),
            pltpu.VMEM((N,), jnp.float32),
            pltpu.VMEM((GCH,), jnp.int32),
            pltpu.VMEM((GCH,), jnp.int32),
            pltpu.VMEM((GCH,), jnp.float32),
            pltpu.VMEM((GCH,), jnp.float32),
            pltpu.VMEM((e_chunks, CH), jnp.int32),
            pltpu.VMEM((CH, D), jnp.float32),
            pltpu.SemaphoreType.DMA,
        ],
    )
    def k(px_h, py_h, pz_h, src_h, dst_h, srcf_h, dstf_h, emb_h, at_h,
          d2_h, vs_h, d2f_h, nf0_h,
          px, py, pz, si, di, d2v, vsv, eidx, ebuf, sem):
        wid = lax.axis_index("s") * NC + lax.axis_index("c")
        base = pl.multiple_of(wid * per_w, 8)

        ebase = pl.multiple_of(wid * e_chunks * CH, 8)
        pltpu.sync_copy(at_h.at[wid], eidx)
        for ck in range(e_chunks):
            pltpu.async_copy(emb_h.at[eidx.at[ck]], ebuf, sem).wait()
            pltpu.sync_copy(ebuf, nf0_h.at[pl.ds(ebase + ck * CH, CH)])

        pltpu.sync_copy(px_h, px)
        pltpu.sync_copy(py_h, py)
        pltpu.sync_copy(pz_h, pz)

        def run(src_ref, dst_ref, out_d2, out_vs):
            def chunk(gc, carry):
                cbase = pl.multiple_of(base + gc * GCH, 8)
                pltpu.sync_copy(src_ref.at[pl.ds(cbase, GCH)], si)
                pltpu.sync_copy(dst_ref.at[pl.ds(cbase, GCH)], di)

                @plsc.parallel_loop(0, iters, 1, unroll=4)
                def body(i):
                    off = pl.multiple_of(i * 16, 8)
                    s16 = si[pl.ds(off, 16)]
                    d16 = di[pl.ds(off, 16)]
                    vx = plsc.load_gather(px, [d16]) - plsc.load_gather(px, [s16])
                    vy = plsc.load_gather(py, [d16]) - plsc.load_gather(py, [s16])
                    vz = plsc.load_gather(pz, [d16]) - plsc.load_gather(pz, [s16])
                    d2v[pl.ds(off, 16)] = vx * vx + vy * vy + vz * vz
                    if out_vs is not None:
                        vsv[pl.ds(off, 16)] = vx + vy + vz
                pltpu.sync_copy(d2v, out_d2.at[pl.ds(cbase, GCH)])
                if out_vs is not None:
                    pltpu.sync_copy(vsv, out_vs.at[pl.ds(cbase, GCH)])
                return carry
            lax.fori_loop(0, g_chunks, chunk, 0)

        run(src_h, dst_h, d2_h, vs_h)
        run(srcf_h, dstf_h, d2f_h, None)

    return k


SBN = 5


def _make_sc_conv():
    per_w = E // NW
    n_chunks = per_w // CH
    rows_per_tile = NACC // NS

    @functools.partial(
        pl.kernel,
        out_type=jax.ShapeDtypeStruct((NC, NACC, D), jnp.float32),
        mesh=plsc.VectorSubcoreMesh(**_MESH),
        scratch_types=[
            pltpu.VMEM_SHARED((NACC, D), jnp.float32),
            pltpu.VMEM((2, CH, D), jnp.float32),
            pltpu.VMEM((2, CH, D), jnp.float32),
            pltpu.VMEM((2, SBN, CH), jnp.int32),
            pltpu.VMEM((2, SBN, CH), jnp.int32),
            pltpu.SemaphoreType.DMA,
            pltpu.SemaphoreType.DMA,
            pltpu.SemaphoreType.DMA,
        ],
    )
    def k(nf_hbm, src_hbm, dst_hbm, ew_hbm, zeros_hbm, out_hbm,
          acc_sh, nfb, ewb, sidx, didx, gsem, esem, ssem):
        sid = lax.axis_index("s")
        cid = lax.axis_index("c")
        wid = sid * NC + cid
        base = pl.multiple_of(wid * per_w, 8)
        tbase = pl.multiple_of(sid * rows_per_tile, 8)
        pltpu.sync_copy(zeros_hbm.at[pl.ds(tbase, rows_per_tile)],
                        acc_sh.at[pl.ds(tbase, rows_per_tile)])

        pltpu.sync_copy(src_hbm.at[wid, 0], sidx.at[0])
        pltpu.sync_copy(dst_hbm.at[wid, 0], didx.at[0])
        plsc.subcore_barrier()
        pltpu.async_copy(nf_hbm.at[sidx.at[0, 0]], nfb.at[0], gsem)
        pltpu.async_copy(ew_hbm.at[pl.ds(base, CH)], ewb.at[0], esem)

        def body(ck, carry):
            slot = lax.rem(ck, 2)
            nxt = lax.rem(ck + 1, 2)
            nb = (ck + 1) // SBN
            nbs = lax.rem(nb, 2)

            @pl.when(ck >= 1)
            def _():
                pltpu.make_async_copy(
                    nfb.at[nxt], acc_sh.at[pl.ds(0, CH)], ssem).wait()

            @pl.when(jnp.logical_and(lax.rem(ck + 1, SBN) == 0,
                                     ck + 1 < n_chunks))
            def _():
                pltpu.sync_copy(src_hbm.at[wid, nb], sidx.at[nbs])
                pltpu.sync_copy(dst_hbm.at[wid, nb], didx.at[nbs])

            @pl.when(ck + 1 < n_chunks)
            def _():
                pltpu.async_copy(
                    nf_hbm.at[sidx.at[nbs, lax.rem(ck + 1, SBN)]],
                    nfb.at[nxt], gsem)
                pltpu.async_copy(ew_hbm.at[pl.ds(base + (ck + 1) * CH, CH)],
                                 ewb.at[nxt], esem)

            pltpu.make_async_copy(nf_hbm.at[sidx.at[0, 0]], nfb.at[slot],
                                  gsem).wait()
            pltpu.make_async_copy(ew_hbm.at[pl.ds(base, CH)], ewb.at[slot],
                                  esem).wait()
            _mul_rows(nfb.at[slot], nfb.at[slot], ewb.at[slot], CH)
            pltpu.make_async_copy(
                nfb.at[slot],
                acc_sh.at[didx.at[lax.rem(ck // SBN, 2), lax.rem(ck, SBN)]],
                ssem).start(add=True)
            return carry

        lax.fori_loop(0, n_chunks, body, 0)
        pltpu.make_async_copy(nfb.at[0], acc_sh.at[pl.ds(0, CH)], ssem).wait()
        plsc.subcore_barrier()
        pltpu.sync_copy(acc_sh.at[pl.ds(tbase, rows_per_tile)],
                        out_hbm.at[cid, pl.ds(tbase, rows_per_tile)])

    return k


def _make_sc_pair_add():
    per_w = E // NW
    n_chunks = per_w // CH

    @functools.partial(
        pl.kernel,
        out_type=jax.ShapeDtypeStruct((E, D), jnp.float32),
        mesh=plsc.VectorSubcoreMesh(**_MESH),
        scratch_types=[
            pltpu.VMEM((2, CH, D), jnp.float32),
            pltpu.VMEM((2, CH, D), jnp.float32),
            pltpu.VMEM((n_chunks, CH), jnp.int32),
            pltpu.VMEM((n_chunks, CH), jnp.int32),
            pltpu.SemaphoreType.DMA,
            pltpu.SemaphoreType.DMA,
            pltpu.SemaphoreType.DMA,
        ],
    )
    def k(g_hbm, src_hbm, dst_hbm, out_hbm, sb, db, sidx, didx,
          s_sem, d_sem, osem):
        wid = lax.axis_index("s") * NC + lax.axis_index("c")
        base = pl.multiple_of(wid * per_w, 8)
        pltpu.sync_copy(src_hbm.at[wid], sidx)
        pltpu.sync_copy(dst_hbm.at[wid], didx)

        pltpu.async_copy(g_hbm.at[sidx.at[0]], sb.at[0], s_sem)
        pltpu.async_copy(g_hbm.at[didx.at[0]], db.at[0], d_sem)

        def body(ck, carry):
            slot = lax.rem(ck, 2)
            nxt = lax.rem(ck + 1, 2)

            @pl.when(jnp.logical_and(ck >= 1, ck + 1 < n_chunks))
            def _():
                pltpu.make_async_copy(
                    sb.at[nxt], out_hbm.at[pl.ds(base, CH)], osem).wait()

            @pl.when(ck + 1 < n_chunks)
            def _():
                pltpu.async_copy(g_hbm.at[sidx.at[ck + 1]], sb.at[nxt], s_sem)
                pltpu.async_copy(g_hbm.at[didx.at[ck + 1]], db.at[nxt], d_sem)

            pltpu.make_async_copy(g_hbm.at[sidx.at[ck]], sb.at[slot],
                                  s_sem).wait()
            pltpu.make_async_copy(g_hbm.at[didx.at[ck]], db.at[slot],
                                  d_sem).wait()
            _add_rows(sb.at[slot], sb.at[slot], db.at[slot], CH)
            pltpu.async_copy(sb.at[slot],
                             out_hbm.at[pl.ds(base + ck * CH, CH)], osem)
            return carry

        lax.fori_loop(0, n_chunks, body, 0)
        pltpu.make_async_copy(sb.at[0], out_hbm.at[pl.ds(base, CH)],
                              osem).wait()
        pltpu.make_async_copy(sb.at[1], out_hbm.at[pl.ds(base, CH)],
                              osem).wait()

    return k


_LOG2E = 1.4426950408889634
_COS_C = (0.0016053627764966202, -0.02539111138418885, 0.2350633717632542,
          -1.3351744534108685, 4.058698262269186, -4.934801388370931,
          0.9999999922898464)


def _rbf_t(d2row, vsrow=None):
    be = d2row.shape[1]
    d2b = jnp.broadcast_to(d2row, (NB, be))
    distb = jnp.sqrt(d2b + 1e-8)
    centers = lax.broadcasted_iota(jnp.int32, (NB, 1), 0).astype(jnp.float32) * (
        CUTOFF / (NB - 1))
    a = distb - centers
    g = jnp.exp2(a * a * (-2.0 * _LOG2E))
    t = jnp.minimum(distb, CUTOFF) * (1.0 / CUTOFF)
    u = t * t
    c = _COS_C[0]
    for coef in _COS_C[1:]:
        c = c * u + coef
    rbf = g * (0.5 * (c + 1.0))
    if vsrow is not None:
        rbf = rbf * (1.0 + jnp.broadcast_to(vsrow, (NB, be)) / (3.0 * distb))
    return rbf


_DN_T = (((0,), (0,)), ((), ()))


def _ew_body(d2_ref, vs_ref, w_ref, ew0_ref, ew1_ref, ew2_ref):
    rbf = _rbf_t(d2_ref[0], vs_ref[0])
    ew0_ref[...] = lax.dot_general(rbf, w_ref[0], _DN_T,
                                   preferred_element_type=jnp.float32)
    ew1_ref[...] = lax.dot_general(rbf, w_ref[1], _DN_T,
                                   preferred_element_type=jnp.float32)
    ew2_ref[...] = lax.dot_general(rbf, w_ref[2], _DN_T,
                                   preferred_element_type=jnp.float32)


def _tc_ew_all(d2, vs, w3, be=2560):
    grid = (E // be,)
    eblk = pl.BlockSpec((be, D), lambda b: (b, 0))
    sd = jax.ShapeDtypeStruct((E, D), jnp.float32)
    return pl.pallas_call(
        _ew_body,
        grid=grid,
        in_specs=[
            pl.BlockSpec((1, 1, be), lambda b: (b, 0, 0)),
            pl.BlockSpec((1, 1, be), lambda b: (b, 0, 0)),
            pl.BlockSpec((3, NB, D), lambda b: (0, 0, 0)),
        ],
        out_specs=(eblk, eblk, eblk),
        out_shape=(sd, sd, sd),
    )(d2.reshape(E // be, 1, be), vs.reshape(E // be, 1, be), w3)


def _update_body(nf_ref, agg_ref, ws_ref, wm_ref, out_ref):
    agg = agg_ref[0] + agg_ref[1]
    h = (jnp.dot(nf_ref[...], ws_ref[...], preferred_element_type=jnp.float32)
         + jnp.dot(agg, wm_ref[...], preferred_element_type=jnp.float32))
    out_ref[...] = _silu(h)


def _update_ext_body(nf_ref, agg_ref, ws_ref, wm_ref, wii_ref, wij_ref,
                     out_ref, g_ref, fii_ref):
    agg = agg_ref[0] + agg_ref[1]
    h = (jnp.dot(nf_ref[...], ws_ref[...], preferred_element_type=jnp.float32)
         + jnp.dot(agg, wm_ref[...], preferred_element_type=jnp.float32))
    nf = _silu(h)
    out_ref[...] = nf
    g_ref[...] = jnp.dot(nf, wij_ref[...], preferred_element_type=jnp.float32)
    t = jnp.dot(nf, wii_ref[...], preferred_element_type=jnp.float32)
    fii_ref[...] = _silu(t)


def _tc_update(nf, aggp, ws, wm, bn=2000):
    grid = (N // bn,)
    blk = pl.BlockSpec((bn, D), lambda b: (b, 0))
    ablk = pl.BlockSpec((NC, bn, D), lambda b: (0, b, 0))
    wblk = pl.BlockSpec((D, D), lambda b: (0, 0))
    return pl.pallas_call(
        _update_body, grid=grid,
        in_specs=[blk, ablk, wblk, wblk],
        out_specs=blk,
        out_shape=jax.ShapeDtypeStruct((N, D), jnp.float32),
    )(nf, aggp, ws, wm)


def _tc_update_ext(nf, aggp, ws, wm, wii, wij, bn=2000):
    grid = (N // bn,)
    blk = pl.BlockSpec((bn, D), lambda b: (b, 0))
    ablk = pl.BlockSpec((NC, bn, D), lambda b: (0, b, 0))
    wblk = pl.BlockSpec((D, D), lambda b: (0, 0))
    sd = jax.ShapeDtypeStruct((N, D), jnp.float32)
    return pl.pallas_call(
        _update_ext_body, grid=grid,
        in_specs=[blk, ablk, wblk, wblk, wblk, wblk],
        out_specs=(blk, blk, blk),
        out_shape=(sd, sd, sd),
    )(nf, aggp, ws, wm, wii, wij)


def _silu(h):
    return h / (1.0 + jnp.exp2(h * (-_LOG2E)))


def _offdiag_body0(d2e_ref, d2o_ref, h2_ref, w2_ref, w4_ref, out_ref):
    rbf2 = jnp.concatenate([_rbf_t(d2e_ref[0]), _rbf_t(d2o_ref[0])], axis=0)
    ew2 = lax.dot_general(rbf2, w2_ref[...], _DN_T,
                          preferred_element_type=jnp.float32)
    t2 = _silu(h2_ref[...]) * ew2
    out_ref[...] = jnp.dot(t2, w4_ref[...], preferred_element_type=jnp.float32)


def _offdiag_body1(d2e_ref, d2o_ref, h2_ref, w2_ref, w4_ref, prev_ref,
                   out_ref):
    rbf2 = jnp.concatenate([_rbf_t(d2e_ref[0]), _rbf_t(d2o_ref[0])], axis=0)
    ew2 = lax.dot_general(rbf2, w2_ref[...], _DN_T,
                          preferred_element_type=jnp.float32)
    t2 = _silu(h2_ref[...]) * ew2
    out_ref[...] = prev_ref[...] + jnp.dot(
        t2, w4_ref[...], preferred_element_type=jnp.float32)


def _tc_offdiag(d2fe, d2fo, h, wrbf, wout, prev=None, be2=1280):
    E2 = E // 2
    grid = (E2 // be2,)
    w2 = jnp.zeros((2 * NB, 2 * D), jnp.float32)
    w2 = w2.at[:NB, :D].set(wrbf).at[NB:, D:].set(wrbf)
    w4 = jnp.zeros((2 * D, 2 * OUT), jnp.float32)
    w4 = w4.at[:D, :OUT].set(wout).at[D:, OUT:].set(wout)
    specs = [
        pl.BlockSpec((1, 1, be2), lambda b: (b, 0, 0)),
        pl.BlockSpec((1, 1, be2), lambda b: (b, 0, 0)),
        pl.BlockSpec((be2, 2 * D), lambda b: (b, 0)),
        pl.BlockSpec((2 * NB, 2 * D), lambda b: (0, 0)),
        pl.BlockSpec((2 * D, 2 * OUT), lambda b: (0, 0)),
    ]
    args = [d2fe.reshape(E2 // be2, 1, be2), d2fo.reshape(E2 // be2, 1, be2),
            h.reshape(E2, 2 * D), w2, w4]
    body = _offdiag_body0
    if prev is not None:
        specs.append(pl.BlockSpec((be2, 2 * OUT), lambda b: (b, 0)))
        args.append(prev)
        body = _offdiag_body1
    return pl.pallas_call(
        body, grid=grid,
        in_specs=specs,
        out_specs=pl.BlockSpec((be2, 2 * OUT), lambda b: (b, 0)),
        out_shape=jax.ShapeDtypeStruct((E2, 2 * OUT), jnp.float32),
    )(*args)


def _diag_body(f0_ref, f1_ref, n0_ref, w_ref, out_ref):
    s = f0_ref[...] + f1_ref[...] + n0_ref[...]
    out_ref[...] = jnp.dot(s, w_ref[...], preferred_element_type=jnp.float32)


def _tc_diag(f0, f1, n0, w, bn=2000):
    grid = (N // bn,)
    blk = pl.BlockSpec((bn, D), lambda b: (b, 0))
    return pl.pallas_call(
        _diag_body, grid=grid,
        in_specs=[blk, blk, blk, pl.BlockSpec((D, OUT), lambda b: (0, 0))],
        out_specs=pl.BlockSpec((bn, OUT), lambda b: (b, 0)),
        out_shape=jax.ShapeDtypeStruct((N, OUT), jnp.float32),
    )(f0, f1, n0, w)


def kernel(at_no, pos, edge_index, edge_index_full, embed_table, conv_Wrbf,
           conv_Wself, conv_Wmsg, trans_Wii, trans_Wrbf, trans_Wij,
           out_Wii, out_Wij):
    f32 = jnp.float32
    src = edge_index[0].astype(jnp.int32)
    dst = edge_index[1].astype(jnp.int32)
    src_f = edge_index_full[0].astype(jnp.int32)
    dst_f = edge_index_full[1].astype(jnp.int32)

    posf = pos.astype(f32)
    zeros_nd = jnp.zeros((NACC, D), f32)
    src4d = src.reshape(NW, -1, SBN, CH)
    dst4d = dst.reshape(NW, -1, SBN, CH)
    srcf3d = src_f.reshape(NW, -1, CH)
    dstf3d = dst_f.reshape(NW, -1, CH)
    at3d = jnp.pad(at_no.astype(jnp.int32), (0, NPAD - N)).reshape(NW, -1, CH)

    d2, vs, d2f, nf0p = _make_sc_geo_embed()(
        posf[:, 0], posf[:, 1], posf[:, 2], src, dst, src_f, dst_f,
        embed_table.astype(f32), at3d)
    nf0 = nf0p[:N]

    ews = _tc_ew_all(d2, vs, conv_Wrbf.astype(f32))
    pair_add = _make_sc_pair_add()
    conv = _make_sc_conv()
    d2fe = d2f[0::2]
    d2fo = d2f[1::2]
    wout = out_Wij.astype(f32)

    aggp = conv(nf0, src4d, dst4d, ews[0], zeros_nd)
    nf = _tc_update(nf0, aggp, conv_Wself[0].astype(f32),
                    conv_Wmsg[0].astype(f32))

    aggp = conv(nf, src4d, dst4d, ews[1], zeros_nd)
    nf, g0, fii0 = _tc_update_ext(
        nf, aggp, conv_Wself[1].astype(f32), conv_Wmsg[1].astype(f32),
        trans_Wii[0].astype(f32), trans_Wij[0].astype(f32))
    h0 = pair_add(g0, srcf3d, dstf3d)

    aggp = conv(nf, src4d, dst4d, ews[2], zeros_nd)
    offd = _tc_offdiag(d2fe, d2fo, h0, trans_Wrbf[0].astype(f32), wout)

    nf, g1, fii1 = _tc_update_ext(
        nf, aggp, conv_Wself[2].astype(f32), conv_Wmsg[2].astype(f32),
        trans_Wii[1].astype(f32), trans_Wij[1].astype(f32))
    h1 = pair_add(g1, srcf3d, dstf3d)
    offd = _tc_offdiag(d2fe, d2fo, h1, trans_Wrbf[1].astype(f32), wout,
                       prev=offd)

    diag = _tc_diag(fii0, fii1, nf0, out_Wii.astype(f32))
    return (diag, offd.reshape(E, OUT))


# speedup vs baseline: 1.2089x; 1.2089x over previous
import functools

import jax
import jax.numpy as jnp
from jax import lax
from jax.experimental import pallas as pl
from jax.experimental.pallas import tpu as pltpu
from jax.experimental.pallas import tpu_sc as plsc

N = 10000
E = 320000
D = 128
NB = 32
OUT = 64
CUTOFF = 5.0

NC = 2
NS = 16
NW = NC * NS
CH = 80
NACC = 10240
NPAD = NW * CH * 4

_MESH = dict(core_axis_name="c", subcore_axis_name="s")
_NOLAYOUT = pltpu.CompilerParams(needs_layout_passes=False)


def _mul_rows(dst_ref, a_ref, b_ref, n_rows):
    @plsc.parallel_loop(0, n_rows, 1, unroll=4)
    def row(r):
        for c8 in range(D // 16):
            s = pl.ds(c8 * 16, 16)
            dst_ref[r, s] = a_ref[r, s] * b_ref[r, s]


def _add_rows(dst_ref, a_ref, b_ref, n_rows):
    @plsc.parallel_loop(0, n_rows, 1, unroll=4)
    def row(r):
        for c8 in range(D // 16):
            s = pl.ds(c8 * 16, 16)
            dst_ref[r, s] = a_ref[r, s] + b_ref[r, s]


def _make_sc_geo_embed():
    per_w = E // NW
    GCH = 2000
    g_chunks = per_w // GCH
    iters = GCH // 16
    e_chunks = NPAD // (NW * CH)

    sd = jax.ShapeDtypeStruct((E,), jnp.float32)

    @functools.partial(
        pl.kernel,
        out_type=(sd, sd, sd, jax.ShapeDtypeStruct((NPAD, D), jnp.float32)),
        mesh=plsc.VectorSubcoreMesh(**_MESH),
        compiler_params=_NOLAYOUT,
        scratch_types=[
            pltpu.VMEM((N,), jnp.float32),
            pltpu.VMEM((N,), jnp.float32),
            pltpu.VMEM((N,), jnp.float32),
            pltpu.VMEM((GCH,), jnp.int32),
            pltpu.VMEM((GCH,), jnp.int32),
            pltpu.VMEM((GCH,), jnp.float32),
            pltpu.VMEM((GCH,), jnp.float32),
            pltpu.VMEM((e_chunks, CH), jnp.int32),
            pltpu.VMEM((CH, D), jnp.float32),
            pltpu.SemaphoreType.DMA,
        ],
    )
    def k(px_h, py_h, pz_h, src_h, dst_h, srcf_h, dstf_h, emb_h, at_h,
          d2_h, vs_h, d2f_h, nf0_h,
          px, py, pz, si, di, d2v, vsv, eidx, ebuf, sem):
        wid = lax.axis_index("s") * NC + lax.axis_index("c")
        base = pl.multiple_of(wid * per_w, 8)

        ebase = pl.multiple_of(wid * e_chunks * CH, 8)
        pltpu.sync_copy(at_h.at[wid], eidx)
        for ck in range(e_chunks):
            pltpu.async_copy(emb_h.at[eidx.at[ck]], ebuf, sem).wait()
            pltpu.sync_copy(ebuf, nf0_h.at[pl.ds(ebase + ck * CH, CH)])

        pltpu.sync_copy(px_h, px)
        pltpu.sync_copy(py_h, py)
        pltpu.sync_copy(pz_h, pz)

        def run(src_ref, dst_ref, out_d2, out_vs):
            def chunk(gc, carry):
                cbase = pl.multiple_of(base + gc * GCH, 8)
                pltpu.sync_copy(src_ref.at[pl.ds(cbase, GCH)], si)
                pltpu.sync_copy(dst_ref.at[pl.ds(cbase, GCH)], di)

                @plsc.parallel_loop(0, iters, 1, unroll=4)
                def body(i):
                    off = pl.multiple_of(i * 16, 8)
                    s16 = si[pl.ds(off, 16)]
                    d16 = di[pl.ds(off, 16)]
                    vx = plsc.load_gather(px, [d16]) - plsc.load_gather(px, [s16])
                    vy = plsc.load_gather(py, [d16]) - plsc.load_gather(py, [s16])
                    vz = plsc.load_gather(pz, [d16]) - plsc.load_gather(pz, [s16])
                    d2v[pl.ds(off, 16)] = vx * vx + vy * vy + vz * vz
                    if out_vs is not None:
                        vsv[pl.ds(off, 16)] = vx + vy + vz
                pltpu.sync_copy(d2v, out_d2.at[pl.ds(cbase, GCH)])
                if out_vs is not None:
                    pltpu.sync_copy(vsv, out_vs.at[pl.ds(cbase, GCH)])
                return carry
            lax.fori_loop(0, g_chunks, chunk, 0)

        run(src_h, dst_h, d2_h, vs_h)
        run(srcf_h, dstf_h, d2f_h, None)

    return k


SBN = 5


def _make_sc_conv():
    per_w = E // NW
    n_chunks = per_w // CH
    rows_per_tile = NACC // NS

    @functools.partial(
        pl.kernel,
        out_type=jax.ShapeDtypeStruct((NC, NACC, D), jnp.float32),
        mesh=plsc.VectorSubcoreMesh(**_MESH),
        scratch_types=[
            pltpu.VMEM_SHARED((NACC, D), jnp.float32),
            pltpu.VMEM((2, CH, D), jnp.float32),
            pltpu.VMEM((2, CH, D), jnp.float32),
            pltpu.VMEM((2, SBN, CH), jnp.int32),
            pltpu.VMEM((2, SBN, CH), jnp.int32),
            pltpu.SemaphoreType.DMA,
            pltpu.SemaphoreType.DMA,
            pltpu.SemaphoreType.DMA,
        ],
    )
    def k(nf_hbm, src_hbm, dst_hbm, ew_hbm, zeros_hbm, out_hbm,
          acc_sh, nfb, ewb, sidx, didx, gsem, esem, ssem):
        sid = lax.axis_index("s")
        cid = lax.axis_index("c")
        wid = sid * NC + cid
        base = pl.multiple_of(wid * per_w, 8)
        tbase = pl.multiple_of(sid * rows_per_tile, 8)
        pltpu.sync_copy(zeros_hbm.at[pl.ds(tbase, rows_per_tile)],
                        acc_sh.at[pl.ds(tbase, rows_per_tile)])

        pltpu.sync_copy(src_hbm.at[wid, 0], sidx.at[0])
        pltpu.sync_copy(dst_hbm.at[wid, 0], didx.at[0])
        plsc.subcore_barrier()
        pltpu.async_copy(nf_hbm.at[sidx.at[0, 0]], nfb.at[0], gsem)
        pltpu.async_copy(ew_hbm.at[pl.ds(base, CH)], ewb.at[0], esem)

        def body(ck, carry):
            slot = lax.rem(ck, 2)
            nxt = lax.rem(ck + 1, 2)
            nb = (ck + 1) // SBN
            nbs = lax.rem(nb, 2)

            @pl.when(ck >= 1)
            def _():
                pltpu.make_async_copy(
                    nfb.at[nxt], acc_sh.at[pl.ds(0, CH)], ssem).wait()

            @pl.when(jnp.logical_and(lax.rem(ck + 1, SBN) == 0,
                                     ck + 1 < n_chunks))
            def _():
                pltpu.sync_copy(src_hbm.at[wid, nb], sidx.at[nbs])
                pltpu.sync_copy(dst_hbm.at[wid, nb], didx.at[nbs])

            @pl.when(ck + 1 < n_chunks)
            def _():
                pltpu.async_copy(
                    nf_hbm.at[sidx.at[nbs, lax.rem(ck + 1, SBN)]],
                    nfb.at[nxt], gsem)
                pltpu.async_copy(ew_hbm.at[pl.ds(base + (ck + 1) * CH, CH)],
                                 ewb.at[nxt], esem)

            pltpu.make_async_copy(nf_hbm.at[sidx.at[0, 0]], nfb.at[slot],
                                  gsem).wait()
            pltpu.make_async_copy(ew_hbm.at[pl.ds(base, CH)], ewb.at[slot],
                                  esem).wait()
            _mul_rows(nfb.at[slot], nfb.at[slot], ewb.at[slot], CH)
            pltpu.make_async_copy(
                nfb.at[slot],
                acc_sh.at[didx.at[lax.rem(ck // SBN, 2), lax.rem(ck, SBN)]],
                ssem).start(add=True)
            return carry

        lax.fori_loop(0, n_chunks, body, 0)
        pltpu.make_async_copy(nfb.at[0], acc_sh.at[pl.ds(0, CH)], ssem).wait()
        plsc.subcore_barrier()
        pltpu.sync_copy(acc_sh.at[pl.ds(tbase, rows_per_tile)],
                        out_hbm.at[cid, pl.ds(tbase, rows_per_tile)])

    return k


def _make_sc_pair_add():
    per_w = E // NW
    n_chunks = per_w // CH

    @functools.partial(
        pl.kernel,
        out_type=jax.ShapeDtypeStruct((E, D), jnp.float32),
        mesh=plsc.VectorSubcoreMesh(**_MESH),
        scratch_types=[
            pltpu.VMEM((2, CH, D), jnp.float32),
            pltpu.VMEM((2, CH, D), jnp.float32),
            pltpu.VMEM((n_chunks, CH), jnp.int32),
            pltpu.VMEM((n_chunks, CH), jnp.int32),
            pltpu.SemaphoreType.DMA,
            pltpu.SemaphoreType.DMA,
            pltpu.SemaphoreType.DMA,
        ],
    )
    def k(g_hbm, src_hbm, dst_hbm, out_hbm, sb, db, sidx, didx,
          s_sem, d_sem, osem):
        wid = lax.axis_index("s") * NC + lax.axis_index("c")
        base = pl.multiple_of(wid * per_w, 8)
        pltpu.sync_copy(src_hbm.at[wid], sidx)
        pltpu.sync_copy(dst_hbm.at[wid], didx)

        pltpu.async_copy(g_hbm.at[sidx.at[0]], sb.at[0], s_sem)
        pltpu.async_copy(g_hbm.at[didx.at[0]], db.at[0], d_sem)

        def body(ck, carry):
            slot = lax.rem(ck, 2)
            nxt = lax.rem(ck + 1, 2)

            @pl.when(jnp.logical_and(ck >= 1, ck + 1 < n_chunks))
            def _():
                pltpu.make_async_copy(
                    sb.at[nxt], out_hbm.at[pl.ds(base, CH)], osem).wait()

            @pl.when(ck + 1 < n_chunks)
            def _():
                pltpu.async_copy(g_hbm.at[sidx.at[ck + 1]], sb.at[nxt], s_sem)
                pltpu.async_copy(g_hbm.at[didx.at[ck + 1]], db.at[nxt], d_sem)

            pltpu.make_async_copy(g_hbm.at[sidx.at[ck]], sb.at[slot],
                                  s_sem).wait()
            pltpu.make_async_copy(g_hbm.at[didx.at[ck]], db.at[slot],
                                  d_sem).wait()
            _add_rows(sb.at[slot], sb.at[slot], db.at[slot], CH)
            pltpu.async_copy(sb.at[slot],
                             out_hbm.at[pl.ds(base + ck * CH, CH)], osem)
            return carry

        lax.fori_loop(0, n_chunks, body, 0)
        pltpu.make_async_copy(sb.at[0], out_hbm.at[pl.ds(base, CH)],
                              osem).wait()
        pltpu.make_async_copy(sb.at[1], out_hbm.at[pl.ds(base, CH)],
                              osem).wait()

    return k


_LOG2E = 1.4426950408889634
_COS_C = (0.0016053627764966202, -0.02539111138418885, 0.2350633717632542,
          -1.3351744534108685, 4.058698262269186, -4.934801388370931,
          0.9999999922898464)


def _rbf_t(d2row, vsrow=None):
    be = d2row.shape[1]
    d2b = jnp.broadcast_to(d2row, (NB, be))
    distb = jnp.sqrt(d2b + 1e-8)
    centers = lax.broadcasted_iota(jnp.int32, (NB, 1), 0).astype(jnp.float32) * (
        CUTOFF / (NB - 1))
    a = distb - centers
    g = jnp.exp2(a * a * (-2.0 * _LOG2E))
    t = jnp.minimum(distb, CUTOFF) * (1.0 / CUTOFF)
    u = t * t
    c = _COS_C[0]
    for coef in _COS_C[1:]:
        c = c * u + coef
    rbf = g * (0.5 * (c + 1.0))
    if vsrow is not None:
        rbf = rbf * (1.0 + jnp.broadcast_to(vsrow, (NB, be)) / (3.0 * distb))
    return rbf


_DN_T = (((0,), (0,)), ((), ()))


def _ew_body(d2_ref, vs_ref, w_ref, ew0_ref, ew1_ref, ew2_ref):
    rbf = _rbf_t(d2_ref[0], vs_ref[0])
    ew0_ref[...] = lax.dot_general(rbf, w_ref[0], _DN_T,
                                   preferred_element_type=jnp.float32)
    ew1_ref[...] = lax.dot_general(rbf, w_ref[1], _DN_T,
                                   preferred_element_type=jnp.float32)
    ew2_ref[...] = lax.dot_general(rbf, w_ref[2], _DN_T,
                                   preferred_element_type=jnp.float32)


def _tc_ew_all(d2, vs, w3, be=2560):
    grid = (E // be,)
    eblk = pl.BlockSpec((be, D), lambda b: (b, 0))
    sd = jax.ShapeDtypeStruct((E, D), jnp.float32)
    return pl.pallas_call(
        _ew_body,
        grid=grid,
        in_specs=[
            pl.BlockSpec((1, 1, be), lambda b: (b, 0, 0)),
            pl.BlockSpec((1, 1, be), lambda b: (b, 0, 0)),
            pl.BlockSpec((3, NB, D), lambda b: (0, 0, 0)),
        ],
        out_specs=(eblk, eblk, eblk),
        out_shape=(sd, sd, sd),
    )(d2.reshape(E // be, 1, be), vs.reshape(E // be, 1, be), w3)


def _update_body(nf_ref, agg_ref, ws_ref, wm_ref, out_ref):
    agg = agg_ref[0] + agg_ref[1]
    h = (jnp.dot(nf_ref[...], ws_ref[...], preferred_element_type=jnp.float32)
         + jnp.dot(agg, wm_ref[...], preferred_element_type=jnp.float32))
    out_ref[...] = _silu(h)


def _update_ext_body(nf_ref, agg_ref, ws_ref, wm_ref, wii_ref, wij_ref,
                     out_ref, g_ref, fii_ref):
    agg = agg_ref[0] + agg_ref[1]
    h = (jnp.dot(nf_ref[...], ws_ref[...], preferred_element_type=jnp.float32)
         + jnp.dot(agg, wm_ref[...], preferred_element_type=jnp.float32))
    nf = _silu(h)
    out_ref[...] = nf
    g_ref[...] = jnp.dot(nf, wij_ref[...], preferred_element_type=jnp.float32)
    t = jnp.dot(nf, wii_ref[...], preferred_element_type=jnp.float32)
    fii_ref[...] = _silu(t)


def _tc_update(nf, aggp, ws, wm, bn=2000):
    grid = (N // bn,)
    blk = pl.BlockSpec((bn, D), lambda b: (b, 0))
    ablk = pl.BlockSpec((NC, bn, D), lambda b: (0, b, 0))
    wblk = pl.BlockSpec((D, D), lambda b: (0, 0))
    return pl.pallas_call(
        _update_body, grid=grid,
        in_specs=[blk, ablk, wblk, wblk],
        out_specs=blk,
        out_shape=jax.ShapeDtypeStruct((N, D), jnp.float32),
    )(nf, aggp, ws, wm)


def _tc_update_ext(nf, aggp, ws, wm, wii, wij, bn=2000):
    grid = (N // bn,)
    blk = pl.BlockSpec((bn, D), lambda b: (b, 0))
    ablk = pl.BlockSpec((NC, bn, D), lambda b: (0, b, 0))
    wblk = pl.BlockSpec((D, D), lambda b: (0, 0))
    sd = jax.ShapeDtypeStruct((N, D), jnp.float32)
    return pl.pallas_call(
        _update_ext_body, grid=grid,
        in_specs=[blk, ablk, wblk, wblk, wblk, wblk],
        out_specs=(blk, blk, blk),
        out_shape=(sd, sd, sd),
    )(nf, aggp, ws, wm, wii, wij)


def _silu(h):
    return h / (1.0 + jnp.exp2(h * (-_LOG2E)))


def _offdiag_body0(d2_ref, h_ref, wrbf_ref, wout_ref, out_ref):
    rbf = _rbf_t(d2_ref[0])
    ew = lax.dot_general(rbf, wrbf_ref[...], _DN_T,
                         preferred_element_type=jnp.float32)
    h = _silu(h_ref[...]) * ew
    out_ref[...] = jnp.dot(h, wout_ref[...], preferred_element_type=jnp.float32)


def _offdiag_body1(d2_ref, h_ref, wrbf_ref, wout_ref, prev_ref, out_ref):
    rbf = _rbf_t(d2_ref[0])
    ew = lax.dot_general(rbf, wrbf_ref[...], _DN_T,
                         preferred_element_type=jnp.float32)
    h = _silu(h_ref[...]) * ew
    out_ref[...] = prev_ref[...] + jnp.dot(
        h, wout_ref[...], preferred_element_type=jnp.float32)


def _tc_offdiag(d2f, h, wrbf, wout, prev=None, be=2560):
    grid = (E // be,)
    specs = [
        pl.BlockSpec((1, 1, be), lambda b: (b, 0, 0)),
        pl.BlockSpec((be, D), lambda b: (b, 0)),
        pl.BlockSpec((NB, D), lambda b: (0, 0)),
        pl.BlockSpec((D, OUT), lambda b: (0, 0)),
    ]
    args = [d2f.reshape(E // be, 1, be), h, wrbf, wout]
    body = _offdiag_body0
    if prev is not None:
        specs.append(pl.BlockSpec((be, OUT), lambda b: (b, 0)))
        args.append(prev)
        body = _offdiag_body1
    return pl.pallas_call(
        body, grid=grid,
        in_specs=specs,
        out_specs=pl.BlockSpec((be, OUT), lambda b: (b, 0)),
        out_shape=jax.ShapeDtypeStruct((E, OUT), jnp.float32),
    )(*args)


def _diag_body(f0_ref, f1_ref, n0_ref, w_ref, out_ref):
    s = f0_ref[...] + f1_ref[...] + n0_ref[...]
    out_ref[...] = jnp.dot(s, w_ref[...], preferred_element_type=jnp.float32)


def _tc_diag(f0, f1, n0, w, bn=2000):
    grid = (N // bn,)
    blk = pl.BlockSpec((bn, D), lambda b: (b, 0))
    return pl.pallas_call(
        _diag_body, grid=grid,
        in_specs=[blk, blk, blk, pl.BlockSpec((D, OUT), lambda b: (0, 0))],
        out_specs=pl.BlockSpec((bn, OUT), lambda b: (b, 0)),
        out_shape=jax.ShapeDtypeStruct((N, OUT), jnp.float32),
    )(f0, f1, n0, w)


def kernel(at_no, pos, edge_index, edge_index_full, embed_table, conv_Wrbf,
           conv_Wself, conv_Wmsg, trans_Wii, trans_Wrbf, trans_Wij,
           out_Wii, out_Wij):
    f32 = jnp.float32
    src = edge_index[0].astype(jnp.int32)
    dst = edge_index[1].astype(jnp.int32)
    src_f = edge_index_full[0].astype(jnp.int32)
    dst_f = edge_index_full[1].astype(jnp.int32)

    posf = pos.astype(f32)
    zeros_nd = jnp.zeros((NACC, D), f32)
    src4d = src.reshape(NW, -1, SBN, CH)
    dst4d = dst.reshape(NW, -1, SBN, CH)
    srcf3d = src_f.reshape(NW, -1, CH)
    dstf3d = dst_f.reshape(NW, -1, CH)
    at3d = jnp.pad(at_no.astype(jnp.int32), (0, NPAD - N)).reshape(NW, -1, CH)

    d2, vs, d2f, nf0p = _make_sc_geo_embed()(
        posf[:, 0], posf[:, 1], posf[:, 2], src, dst, src_f, dst_f,
        embed_table.astype(f32), at3d)
    nf0 = nf0p[:N]

    ews = _tc_ew_all(d2, vs, conv_Wrbf.astype(f32))
    pair_add = _make_sc_pair_add()
    conv = _make_sc_conv()
    wout = out_Wij.astype(f32)

    aggp = conv(nf0, src4d, dst4d, ews[0], zeros_nd)
    nf = _tc_update(nf0, aggp, conv_Wself[0].astype(f32),
                    conv_Wmsg[0].astype(f32))

    aggp = conv(nf, src4d, dst4d, ews[1], zeros_nd)
    nf, g0, fii0 = _tc_update_ext(
        nf, aggp, conv_Wself[1].astype(f32), conv_Wmsg[1].astype(f32),
        trans_Wii[0].astype(f32), trans_Wij[0].astype(f32))
    h0 = pair_add(g0, srcf3d, dstf3d)

    aggp = conv(nf, src4d, dst4d, ews[2], zeros_nd)
    offd = _tc_offdiag(d2f, h0, trans_Wrbf[0].astype(f32), wout)

    nf, g1, fii1 = _tc_update_ext(
        nf, aggp, conv_Wself[2].astype(f32), conv_Wmsg[2].astype(f32),
        trans_Wii[1].astype(f32), trans_Wij[1].astype(f32))
    h1 = pair_add(g1, srcf3d, dstf3d)
    offd = _tc_offdiag(d2f, h1, trans_Wrbf[1].astype(f32), wout, prev=offd)

    diag = _tc_diag(fii0, fii1, nf0, out_Wii.astype(f32))
    return (diag, offd)


# speedup vs baseline: 1.2224x; 1.0112x over previous
import functools

import jax
import jax.numpy as jnp
from jax import lax
from jax.experimental import pallas as pl
from jax.experimental.pallas import tpu as pltpu
from jax.experimental.pallas import tpu_sc as plsc

N = 10000
E = 320000
D = 128
NB = 32
OUT = 64
CUTOFF = 5.0

NC = 2
NS = 16
NW = NC * NS
CH = 80
NACC = 10240
NPAD = NW * CH * 4

_MESH = dict(core_axis_name="c", subcore_axis_name="s")
_NOLAYOUT = pltpu.CompilerParams(needs_layout_passes=False)


def _mul_rows(dst_ref, a_ref, b_ref, n_rows):
    @plsc.parallel_loop(0, n_rows, 1, unroll=4)
    def row(r):
        for c8 in range(D // 16):
            s = pl.ds(c8 * 16, 16)
            dst_ref[r, s] = a_ref[r, s] * b_ref[r, s]


def _add_rows(dst_ref, a_ref, b_ref, n_rows):
    @plsc.parallel_loop(0, n_rows, 1, unroll=4)
    def row(r):
        for c8 in range(D // 16):
            s = pl.ds(c8 * 16, 16)
            dst_ref[r, s] = a_ref[r, s] + b_ref[r, s]


def _make_sc_geo_embed():
    per_w = E // NW
    GCH = 2000
    g_chunks = per_w // GCH
    iters = GCH // 16
    e_chunks = NPAD // (NW * CH)

    sd = jax.ShapeDtypeStruct((E,), jnp.float32)

    @functools.partial(
        pl.kernel,
        out_type=(sd, sd, sd, jax.ShapeDtypeStruct((NPAD, D), jnp.float32)),
        mesh=plsc.VectorSubcoreMesh(**_MESH),
        compiler_params=_NOLAYOUT,
        scratch_types=[
            pltpu.VMEM((N,), jnp.float32),
            pltpu.VMEM((N,), jnp.float32),
            pltpu.VMEM((N,), jnp.float32),
            pltpu.VMEM((GCH,), jnp.int32),
            pltpu.VMEM((GCH,), jnp.int32),
            pltpu.VMEM((GCH,), jnp.float32),
            pltpu.VMEM((GCH,), jnp.float32),
            pltpu.VMEM((e_chunks, CH), jnp.int32),
            pltpu.VMEM((CH, D), jnp.float32),
            pltpu.SemaphoreType.DMA,
        ],
    )
    def k(px_h, py_h, pz_h, src_h, dst_h, srcf_h, dstf_h, emb_h, at_h,
          d2_h, vs_h, d2f_h, nf0_h,
          px, py, pz, si, di, d2v, vsv, eidx, ebuf, sem):
        wid = lax.axis_index("s") * NC + lax.axis_index("c")
        base = pl.multiple_of(wid * per_w, 8)

        ebase = pl.multiple_of(wid * e_chunks * CH, 8)
        pltpu.sync_copy(at_h.at[wid], eidx)
        for ck in range(e_chunks):
            pltpu.async_copy(emb_h.at[eidx.at[ck]], ebuf, sem).wait()
            pltpu.sync_copy(ebuf, nf0_h.at[pl.ds(ebase + ck * CH, CH)])

        pltpu.sync_copy(px_h, px)
        pltpu.sync_copy(py_h, py)
        pltpu.sync_copy(pz_h, pz)

        def run(src_ref, dst_ref, out_d2, out_vs):
            def chunk(gc, carry):
                cbase = pl.multiple_of(base + gc * GCH, 8)
                pltpu.sync_copy(src_ref.at[pl.ds(cbase, GCH)], si)
                pltpu.sync_copy(dst_ref.at[pl.ds(cbase, GCH)], di)

                @plsc.parallel_loop(0, iters, 1, unroll=4)
                def body(i):
                    off = pl.multiple_of(i * 16, 8)
                    s16 = si[pl.ds(off, 16)]
                    d16 = di[pl.ds(off, 16)]
                    vx = plsc.load_gather(px, [d16]) - plsc.load_gather(px, [s16])
                    vy = plsc.load_gather(py, [d16]) - plsc.load_gather(py, [s16])
                    vz = plsc.load_gather(pz, [d16]) - plsc.load_gather(pz, [s16])
                    d2v[pl.ds(off, 16)] = vx * vx + vy * vy + vz * vz
                    if out_vs is not None:
                        vsv[pl.ds(off, 16)] = vx + vy + vz
                pltpu.sync_copy(d2v, out_d2.at[pl.ds(cbase, GCH)])
                if out_vs is not None:
                    pltpu.sync_copy(vsv, out_vs.at[pl.ds(cbase, GCH)])
                return carry
            lax.fori_loop(0, g_chunks, chunk, 0)

        run(src_h, dst_h, d2_h, vs_h)
        run(srcf_h, dstf_h, d2f_h, None)

    return k


SBN = 5


def _make_sc_conv():
    per_w = E // NW
    n_chunks = per_w // CH
    rows_per_tile = NACC // NS

    @functools.partial(
        pl.kernel,
        out_type=jax.ShapeDtypeStruct((NC, NACC, D), jnp.float32),
        mesh=plsc.VectorSubcoreMesh(**_MESH),
        scratch_types=[
            pltpu.VMEM_SHARED((NACC, D), jnp.float32),
            pltpu.VMEM((2, CH, D), jnp.float32),
            pltpu.VMEM((2, CH, D), jnp.float32),
            pltpu.VMEM((2, SBN, CH), jnp.int32),
            pltpu.VMEM((2, SBN, CH), jnp.int32),
            pltpu.SemaphoreType.DMA,
            pltpu.SemaphoreType.DMA,
            pltpu.SemaphoreType.DMA,
        ],
    )
    def k(nf_hbm, src_hbm, dst_hbm, ew_hbm, zeros_hbm, out_hbm,
          acc_sh, nfb, ewb, sidx, didx, gsem, esem, ssem):
        sid = lax.axis_index("s")
        cid = lax.axis_index("c")
        wid = sid * NC + cid
        base = pl.multiple_of(wid * per_w, 8)
        tbase = pl.multiple_of(sid * rows_per_tile, 8)
        pltpu.sync_copy(zeros_hbm.at[pl.ds(tbase, rows_per_tile)],
                        acc_sh.at[pl.ds(tbase, rows_per_tile)])

        pltpu.sync_copy(src_hbm.at[wid, 0], sidx.at[0])
        pltpu.sync_copy(dst_hbm.at[wid, 0], didx.at[0])
        plsc.subcore_barrier()
        pltpu.async_copy(nf_hbm.at[sidx.at[0, 0]], nfb.at[0], gsem)
        pltpu.async_copy(ew_hbm.at[pl.ds(base, CH)], ewb.at[0], esem)

        def body(ck, carry):
            slot = lax.rem(ck, 2)
            nxt = lax.rem(ck + 1, 2)
            nb = (ck + 1) // SBN
            nbs = lax.rem(nb, 2)

            @pl.when(ck >= 1)
            def _():
                pltpu.make_async_copy(
                    nfb.at[nxt], acc_sh.at[pl.ds(0, CH)], ssem).wait()

            @pl.when(jnp.logical_and(lax.rem(ck + 1, SBN) == 0,
                                     ck + 1 < n_chunks))
            def _():
                pltpu.sync_copy(src_hbm.at[wid, nb], sidx.at[nbs])
                pltpu.sync_copy(dst_hbm.at[wid, nb], didx.at[nbs])

            @pl.when(ck + 1 < n_chunks)
            def _():
                pltpu.async_copy(
                    nf_hbm.at[sidx.at[nbs, lax.rem(ck + 1, SBN)]],
                    nfb.at[nxt], gsem)
                pltpu.async_copy(ew_hbm.at[pl.ds(base + (ck + 1) * CH, CH)],
                                 ewb.at[nxt], esem)

            pltpu.make_async_copy(nf_hbm.at[sidx.at[0, 0]], nfb.at[slot],
                                  gsem).wait()
            pltpu.make_async_copy(ew_hbm.at[pl.ds(base, CH)], ewb.at[slot],
                                  esem).wait()
            _mul_rows(nfb.at[slot], nfb.at[slot], ewb.at[slot], CH)
            pltpu.make_async_copy(
                nfb.at[slot],
                acc_sh.at[didx.at[lax.rem(ck // SBN, 2), lax.rem(ck, SBN)]],
                ssem).start(add=True)
            return carry

        lax.fori_loop(0, n_chunks, body, 0)
        pltpu.make_async_copy(nfb.at[0], acc_sh.at[pl.ds(0, CH)], ssem).wait()
        plsc.subcore_barrier()
        pltpu.sync_copy(acc_sh.at[pl.ds(tbase, rows_per_tile)],
                        out_hbm.at[cid, pl.ds(tbase, rows_per_tile)])

    return k


def _make_sc_pair_add():
    per_w = E // NW
    n_chunks = per_w // CH

    @functools.partial(
        pl.kernel,
        out_type=jax.ShapeDtypeStruct((E, D), jnp.float32),
        mesh=plsc.VectorSubcoreMesh(**_MESH),
        scratch_types=[
            pltpu.VMEM((2, CH, D), jnp.float32),
            pltpu.VMEM((2, CH, D), jnp.float32),
            pltpu.VMEM((n_chunks, CH), jnp.int32),
            pltpu.VMEM((n_chunks, CH), jnp.int32),
            pltpu.SemaphoreType.DMA,
            pltpu.SemaphoreType.DMA,
            pltpu.SemaphoreType.DMA,
        ],
    )
    def k(g_hbm, src_hbm, dst_hbm, out_hbm, sb, db, sidx, didx,
          s_sem, d_sem, osem):
        wid = lax.axis_index("s") * NC + lax.axis_index("c")
        base = pl.multiple_of(wid * per_w, 8)
        pltpu.sync_copy(src_hbm.at[wid], sidx)
        pltpu.sync_copy(dst_hbm.at[wid], didx)

        pltpu.async_copy(g_hbm.at[sidx.at[0]], sb.at[0], s_sem)
        pltpu.async_copy(g_hbm.at[didx.at[0]], db.at[0], d_sem)

        def body(ck, carry):
            slot = lax.rem(ck, 2)
            nxt = lax.rem(ck + 1, 2)

            @pl.when(jnp.logical_and(ck >= 1, ck + 1 < n_chunks))
            def _():
                pltpu.make_async_copy(
                    sb.at[nxt], out_hbm.at[pl.ds(base, CH)], osem).wait()

            @pl.when(ck + 1 < n_chunks)
            def _():
                pltpu.async_copy(g_hbm.at[sidx.at[ck + 1]], sb.at[nxt], s_sem)
                pltpu.async_copy(g_hbm.at[didx.at[ck + 1]], db.at[nxt], d_sem)

            pltpu.make_async_copy(g_hbm.at[sidx.at[ck]], sb.at[slot],
                                  s_sem).wait()
            pltpu.make_async_copy(g_hbm.at[didx.at[ck]], db.at[slot],
                                  d_sem).wait()
            _add_rows(sb.at[slot], sb.at[slot], db.at[slot], CH)
            pltpu.async_copy(sb.at[slot],
                             out_hbm.at[pl.ds(base + ck * CH, CH)], osem)
            return carry

        lax.fori_loop(0, n_chunks, body, 0)
        pltpu.make_async_copy(sb.at[0], out_hbm.at[pl.ds(base, CH)],
                              osem).wait()
        pltpu.make_async_copy(sb.at[1], out_hbm.at[pl.ds(base, CH)],
                              osem).wait()

    return k


_LOG2E = 1.4426950408889634
_COS_C = (0.0016053627764966202, -0.02539111138418885, 0.2350633717632542,
          -1.3351744534108685, 4.058698262269186, -4.934801388370931,
          0.9999999922898464)


def _rbf_t(d2row, vsrow=None):
    be = d2row.shape[1]
    d2b = jnp.broadcast_to(d2row, (NB, be))
    distb = jnp.sqrt(d2b + 1e-8)
    centers = lax.broadcasted_iota(jnp.int32, (NB, 1), 0).astype(jnp.float32) * (
        CUTOFF / (NB - 1))
    a = distb - centers
    g = jnp.exp2(a * a * (-2.0 * _LOG2E))
    t = jnp.minimum(distb, CUTOFF) * (1.0 / CUTOFF)
    u = t * t
    c = _COS_C[0]
    for coef in _COS_C[1:]:
        c = c * u + coef
    rbf = g * (0.5 * (c + 1.0))
    if vsrow is not None:
        rbf = rbf * (1.0 + jnp.broadcast_to(vsrow, (NB, be)) / (3.0 * distb))
    return rbf


_DN_T = (((0,), (0,)), ((), ()))


def _ew_body(d2_ref, vs_ref, w_ref, ew_ref):
    rbf = _rbf_t(d2_ref[0], vs_ref[0])
    ew_ref[...] = lax.dot_general(rbf, w_ref[...], _DN_T,
                                  preferred_element_type=jnp.float32)


def _tc_ew(d2, vs, w, be=2560):
    grid = (E // be,)
    return pl.pallas_call(
        _ew_body,
        grid=grid,
        in_specs=[
            pl.BlockSpec((1, 1, be), lambda b: (b, 0, 0)),
            pl.BlockSpec((1, 1, be), lambda b: (b, 0, 0)),
            pl.BlockSpec((NB, D), lambda b: (0, 0)),
        ],
        out_specs=pl.BlockSpec((be, D), lambda b: (b, 0)),
        out_shape=jax.ShapeDtypeStruct((E, D), jnp.float32),
    )(d2.reshape(E // be, 1, be), vs.reshape(E // be, 1, be), w)


def _update_body(nf_ref, agg_ref, ws_ref, wm_ref, out_ref):
    agg = agg_ref[0] + agg_ref[1]
    h = (jnp.dot(nf_ref[...], ws_ref[...], preferred_element_type=jnp.float32)
         + jnp.dot(agg, wm_ref[...], preferred_element_type=jnp.float32))
    out_ref[...] = _silu(h)


def _update_ext_body(nf_ref, agg_ref, ws_ref, wm_ref, wii_ref, wij_ref,
                     out_ref, g_ref, fii_ref):
    agg = agg_ref[0] + agg_ref[1]
    h = (jnp.dot(nf_ref[...], ws_ref[...], preferred_element_type=jnp.float32)
         + jnp.dot(agg, wm_ref[...], preferred_element_type=jnp.float32))
    nf = _silu(h)
    out_ref[...] = nf
    g_ref[...] = jnp.dot(nf, wij_ref[...], preferred_element_type=jnp.float32)
    t = jnp.dot(nf, wii_ref[...], preferred_element_type=jnp.float32)
    fii_ref[...] = _silu(t)


def _tc_update(nf, aggp, ws, wm, bn=2000):
    grid = (N // bn,)
    blk = pl.BlockSpec((bn, D), lambda b: (b, 0))
    ablk = pl.BlockSpec((NC, bn, D), lambda b: (0, b, 0))
    wblk = pl.BlockSpec((D, D), lambda b: (0, 0))
    return pl.pallas_call(
        _update_body, grid=grid,
        in_specs=[blk, ablk, wblk, wblk],
        out_specs=blk,
        out_shape=jax.ShapeDtypeStruct((N, D), jnp.float32),
    )(nf, aggp, ws, wm)


def _tc_update_ext(nf, aggp, ws, wm, wii, wij, bn=2000):
    grid = (N // bn,)
    blk = pl.BlockSpec((bn, D), lambda b: (b, 0))
    ablk = pl.BlockSpec((NC, bn, D), lambda b: (0, b, 0))
    wblk = pl.BlockSpec((D, D), lambda b: (0, 0))
    sd = jax.ShapeDtypeStruct((N, D), jnp.float32)
    return pl.pallas_call(
        _update_ext_body, grid=grid,
        in_specs=[blk, ablk, wblk, wblk, wblk, wblk],
        out_specs=(blk, blk, blk),
        out_shape=(sd, sd, sd),
    )(nf, aggp, ws, wm, wii, wij)


def _silu(h):
    return h / (1.0 + jnp.exp2(h * (-_LOG2E)))


def _offdiag_body0(d2_ref, h_ref, wrbf_ref, wout_ref, out_ref):
    rbf = _rbf_t(d2_ref[0])
    ew = lax.dot_general(rbf, wrbf_ref[...], _DN_T,
                         preferred_element_type=jnp.float32)
    h = _silu(h_ref[...]) * ew
    out_ref[...] = jnp.dot(h, wout_ref[...], preferred_element_type=jnp.float32)


def _offdiag_body1(d2_ref, h_ref, wrbf_ref, wout_ref, prev_ref, out_ref):
    rbf = _rbf_t(d2_ref[0])
    ew = lax.dot_general(rbf, wrbf_ref[...], _DN_T,
                         preferred_element_type=jnp.float32)
    h = _silu(h_ref[...]) * ew
    out_ref[...] = prev_ref[...] + jnp.dot(
        h, wout_ref[...], preferred_element_type=jnp.float32)


def _tc_offdiag(d2f, h, wrbf, wout, prev=None, be=2560):
    grid = (E // be,)
    specs = [
        pl.BlockSpec((1, 1, be), lambda b: (b, 0, 0)),
        pl.BlockSpec((be, D), lambda b: (b, 0)),
        pl.BlockSpec((NB, D), lambda b: (0, 0)),
        pl.BlockSpec((D, OUT), lambda b: (0, 0)),
    ]
    args = [d2f.reshape(E // be, 1, be), h, wrbf, wout]
    body = _offdiag_body0
    if prev is not None:
        specs.append(pl.BlockSpec((be, OUT), lambda b: (b, 0)))
        args.append(prev)
        body = _offdiag_body1
    return pl.pallas_call(
        body, grid=grid,
        in_specs=specs,
        out_specs=pl.BlockSpec((be, OUT), lambda b: (b, 0)),
        out_shape=jax.ShapeDtypeStruct((E, OUT), jnp.float32),
    )(*args)


def _diag_body(f0_ref, f1_ref, n0_ref, w_ref, out_ref):
    s = f0_ref[...] + f1_ref[...] + n0_ref[...]
    out_ref[...] = jnp.dot(s, w_ref[...], preferred_element_type=jnp.float32)


def _tc_diag(f0, f1, n0, w, bn=2000):
    grid = (N // bn,)
    blk = pl.BlockSpec((bn, D), lambda b: (b, 0))
    return pl.pallas_call(
        _diag_body, grid=grid,
        in_specs=[blk, blk, blk, pl.BlockSpec((D, OUT), lambda b: (0, 0))],
        out_specs=pl.BlockSpec((bn, OUT), lambda b: (b, 0)),
        out_shape=jax.ShapeDtypeStruct((N, OUT), jnp.float32),
    )(f0, f1, n0, w)


def kernel(at_no, pos, edge_index, edge_index_full, embed_table, conv_Wrbf,
           conv_Wself, conv_Wmsg, trans_Wii, trans_Wrbf, trans_Wij,
           out_Wii, out_Wij):
    f32 = jnp.float32
    src = edge_index[0].astype(jnp.int32)
    dst = edge_index[1].astype(jnp.int32)
    src_f = edge_index_full[0].astype(jnp.int32)
    dst_f = edge_index_full[1].astype(jnp.int32)

    posf = pos.astype(f32)
    zeros_nd = jnp.zeros((NACC, D), f32)
    src4d = src.reshape(NW, -1, SBN, CH)
    dst4d = dst.reshape(NW, -1, SBN, CH)
    srcf3d = src_f.reshape(NW, -1, CH)
    dstf3d = dst_f.reshape(NW, -1, CH)
    at3d = jnp.pad(at_no.astype(jnp.int32), (0, NPAD - N)).reshape(NW, -1, CH)

    d2, vs, d2f, nf0p = _make_sc_geo_embed()(
        posf[:, 0], posf[:, 1], posf[:, 2], src, dst, src_f, dst_f,
        embed_table.astype(f32), at3d)
    nf0 = nf0p[:N]

    w3 = conv_Wrbf.astype(f32)
    pair_add = _make_sc_pair_add()
    conv = _make_sc_conv()
    wout = out_Wij.astype(f32)

    ew0 = _tc_ew(d2, vs, w3[0])
    aggp = conv(nf0, src4d, dst4d, ew0, zeros_nd)
    ew1 = _tc_ew(d2, vs, w3[1])
    ew2 = _tc_ew(d2, vs, w3[2])
    nf = _tc_update(nf0, aggp, conv_Wself[0].astype(f32),
                    conv_Wmsg[0].astype(f32))

    aggp = conv(nf, src4d, dst4d, ew1, zeros_nd)
    nf, g0, fii0 = _tc_update_ext(
        nf, aggp, conv_Wself[1].astype(f32), conv_Wmsg[1].astype(f32),
        trans_Wii[0].astype(f32), trans_Wij[0].astype(f32))
    h0 = pair_add(g0, srcf3d, dstf3d)

    aggp = conv(nf, src4d, dst4d, ew2, zeros_nd)
    offd = _tc_offdiag(d2f, h0, trans_Wrbf[0].astype(f32), wout)

    nf, g1, fii1 = _tc_update_ext(
        nf, aggp, conv_Wself[2].astype(f32), conv_Wmsg[2].astype(f32),
        trans_Wii[1].astype(f32), trans_Wij[1].astype(f32))
    h1 = pair_add(g1, srcf3d, dstf3d)
    offd = _tc_offdiag(d2f, h1, trans_Wrbf[1].astype(f32), wout, prev=offd)

    diag = _tc_diag(fii0, fii1, nf0, out_Wii.astype(f32))
    return (diag, offd)
